# baseline (speedup 1.0000x reference)
import functools

import jax
import jax.numpy as jnp
from jax import lax
from jax.experimental import pallas as pl
from jax.experimental.pallas import tpu as pltpu
from jax.experimental.pallas import tpu_sc as plsc

E = 8
T = 2048
D = 768
H = 512
COEF = 0.01
NEG = -1e30
TR = 128
SORT_ROWS = 5120
NTS = SORT_ROWS // TR
NSH = T // TR
NGRID = NTS + NSH
BUF_ROWS = SORT_ROWS + T


def _cumsum_sub_excl(a):
    n = a.shape[0]
    incl = a
    s = 1
    while s < n:
        incl = incl + jnp.pad(incl, ((s, 0), (0, 0)))[:n]
        s *= 2
    return incl - a


def _cumsum_lane_excl(a):
    n = a.shape[1]
    a = jnp.pad(a, ((0, 0), (1, 0)))[:, :n]
    s = 1
    while s < n:
        a = a + jnp.pad(a, ((0, 0), (s, 0)))[:, :n]
        s *= 2
    return a


def _plan_body(x_ref, wr_ref, aux_ref, posn_ref, pbb_ref, tiles_ref):
    x = x_ref[...]
    wr = wr_ref[...]
    logits = lax.dot_general(x, wr, (((1,), (1,)), ((), ())),
                             preferred_element_type=jnp.float32)
    lane = lax.broadcasted_iota(jnp.int32, (T, E), 1)

    m = jnp.max(logits, axis=1, keepdims=True)
    ex = jnp.exp(logits - m)
    probs = ex / jnp.sum(ex, axis=1, keepdims=True)

    i1 = jnp.argmax(logits, axis=1).reshape(T, 1)
    oh1 = (lane == i1).astype(jnp.float32)
    p1 = jnp.max(probs, axis=1, keepdims=True)
    logits2 = jnp.where(lane == i1, NEG, logits)
    i2 = jnp.argmax(logits2, axis=1).reshape(T, 1)
    oh2 = (lane == i2).astype(jnp.float32)
    p2 = jnp.max(jnp.where(lane == i1, NEG, probs), axis=1, keepdims=True)

    density = jnp.mean(oh1, axis=0, keepdims=True)
    rpm = jnp.mean(probs, axis=0, keepdims=True)
    aux = COEF * jnp.sum(density * rpm) * E
    aux_ref[...] = jnp.full((8, 128), aux, dtype=jnp.float32)

    c0 = _cumsum_sub_excl(oh1)
    c1 = _cumsum_sub_excl(oh2)
    total0 = jnp.sum(oh1, axis=0, keepdims=True)
    total1 = jnp.sum(oh2, axis=0, keepdims=True)
    counts = total0 + total1
    cnt_al = jnp.floor((counts + (TR - 1)) / TR) * TR
    base = _cumsum_lane_excl(cnt_al)

    rank0 = jnp.sum(c0 * oh1, axis=1, keepdims=True)
    rank1 = jnp.sum((c1 + total0) * oh2, axis=1, keepdims=True)
    base0 = jnp.sum(base * oh1, axis=1, keepdims=True)
    base1 = jnp.sum(base * oh2, axis=1, keepdims=True)
    pos0f = base0 + rank0
    pos1f = base1 + rank1

    posn_ref[...] = (pos0f * (lane == 0) + pos1f * (lane == 1)).astype(jnp.int32)

    lane128 = lax.broadcasted_iota(jnp.int32, (T, 128), 1)
    pbb_ref[...] = jnp.where(lane128 < 16, p1,
                             jnp.where(lane128 < 32, p2, 0.0))

    s_i = lax.broadcasted_iota(jnp.int32, (64, E), 0).astype(jnp.float32) * TR
    hit = (s_i >= base).astype(jnp.float32)
    te_col = jnp.sum(hit, axis=1, keepdims=True) - 1.0
    total_al = jnp.sum(cnt_al, axis=1, keepdims=True)
    tv_col = (s_i[:, 0:1] < total_al).astype(jnp.float32)
    row_f = lax.broadcasted_iota(jnp.int32, (64, 1), 0).astype(jnp.float32)
    xi_col = jnp.where(tv_col > 0.0, row_f, float(NTS - 1))
    lane8b = lax.broadcasted_iota(jnp.int32, (64, 8), 1)
    tiles_ref[...] = (te_col * (lane8b == 0) + tv_col * (lane8b == 1)
                      + xi_col * (lane8b == 2)).astype(jnp.int32)


def _mlp(x, w1, w3, w2):
    h1 = lax.dot_general(x, w1, (((1,), (1,)), ((), ())),
                         preferred_element_type=jnp.float32)
    h3 = lax.dot_general(x, w3, (((1,), (1,)), ((), ())),
                         preferred_element_type=jnp.float32)
    g = h1 * jax.nn.sigmoid(h1) * h3
    return lax.dot_general(g, w2, (((1,), (1,)), ((), ())),
                           preferred_element_type=jnp.float32)


def _moe_body(tr_ref, xs_ref, xf_ref, w1_ref, w3_ref, w2_ref,
              sw1_ref, sw3_ref, sw2_ref, out_ref):
    i = pl.program_id(0)

    @pl.when(i >= NTS)
    def _():
        out_ref[...] = _mlp(xf_ref[...], sw1_ref[0], sw3_ref[0], sw2_ref[0])

    @pl.when((i < NTS) & (tr_ref[i, 1] == 1))
    def _():
        out_ref[...] = _mlp(xs_ref[...], w1_ref[0], w3_ref[0], w2_ref[0])


def _scatter_body(x_hbm, pos0_hbm, pos1_hbm, xs_hbm, idx0_v, idx1_v,
                  rows_v, sem):
    wid = lax.axis_index("s") * 2 + lax.axis_index("c")
    tok0 = wid * 64
    pltpu.sync_copy(pos0_hbm.at[pl.ds(tok0, 64)], idx0_v)
    pltpu.sync_copy(pos1_hbm.at[pl.ds(tok0, 64)], idx1_v)
    pltpu.sync_copy(x_hbm.at[pl.ds(tok0, 64)], rows_v)
    cp0 = pltpu.async_copy(rows_v, xs_hbm.at[idx0_v], sem)
    cp1 = pltpu.async_copy(rows_v, xs_hbm.at[idx1_v], sem)
    cp0.wait()
    cp1.wait()


def _combine_body(buf_hbm, pos0_hbm, pos1_hbm, pbb_hbm, out_hbm,
                  idx0a_v, idx1a_v, idx0b_v, idx1b_v,
                  r0a_v, r1a_v, r0b_v, r1b_v, acc_v, pb_v, sem_a, sem_b):
    wid = lax.axis_index("s") * 2 + lax.axis_index("c")
    tok0 = wid * 64
    pltpu.sync_copy(pos0_hbm.at[pl.ds(tok0, 32)], idx0a_v)
    pltpu.sync_copy(pos1_hbm.at[pl.ds(tok0, 32)], idx1a_v)
    pltpu.sync_copy(pos0_hbm.at[pl.ds(tok0 + 32, 32)], idx0b_v)
    pltpu.sync_copy(pos1_hbm.at[pl.ds(tok0 + 32, 32)], idx1b_v)
    cps = [pltpu.async_copy(buf_hbm.at[idx0a_v], r0a_v, sem_a),
           pltpu.async_copy(buf_hbm.at[idx1a_v], r1a_v, sem_a),
           pltpu.async_copy(buf_hbm.at[idx0b_v], r0b_v, sem_b),
           pltpu.async_copy(buf_hbm.at[idx1b_v], r1b_v, sem_b)]

    for b, r0_v, r1_v in ((0, r0a_v, r1a_v), (1, r0b_v, r1b_v)):
        t0 = tok0 + b * 32
        pltpu.sync_copy(buf_hbm.at[pl.ds(SORT_ROWS + t0, 32)], acc_v)
        pltpu.sync_copy(pbb_hbm.at[pl.ds(t0, 32)], pb_v)
        cps[2 * b].wait()
        cps[2 * b + 1].wait()

        def body(it, carry):
            for dt in range(4):
                t = it * 4 + dt
                a1 = pb_v[t, pl.ds(0, 16)]
                a2 = pb_v[t, pl.ds(16, 16)]
                for ch in range(48):
                    sl = pl.ds(ch * 16, 16)
                    acc_v[t, sl] = (acc_v[t, sl] + a1 * r0_v[t, sl]
                                    + a2 * r1_v[t, sl])
            return carry

        lax.fori_loop(0, 8, body, 0)
        pltpu.sync_copy(acc_v, out_hbm.at[pl.ds(t0, 32)])


@functools.lru_cache(maxsize=1)
def _sc_kernels():
    mesh = plsc.VectorSubcoreMesh(core_axis_name="c", subcore_axis_name="s")
    scatter_k = pl.kernel(
        _scatter_body,
        out_type=jax.ShapeDtypeStruct((SORT_ROWS, D), jnp.float32),
        mesh=mesh,
        scratch_types=[
            pltpu.VMEM((64,), jnp.int32),
            pltpu.VMEM((64,), jnp.int32),
            pltpu.VMEM((64, D), jnp.float32),
            pltpu.SemaphoreType.DMA,
        ],
    )
    combine_k = pl.kernel(
        _combine_body,
        out_type=jax.ShapeDtypeStruct((T, D), jnp.float32),
        mesh=mesh,
        scratch_types=[
            pltpu.VMEM((32,), jnp.int32),
            pltpu.VMEM((32,), jnp.int32),
            pltpu.VMEM((32,), jnp.int32),
            pltpu.VMEM((32,), jnp.int32),
            pltpu.VMEM((32, D), jnp.float32),
            pltpu.VMEM((32, D), jnp.float32),
            pltpu.VMEM((32, D), jnp.float32),
            pltpu.VMEM((32, D), jnp.float32),
            pltpu.VMEM((32, D), jnp.float32),
            pltpu.VMEM((32, 128), jnp.float32),
            pltpu.SemaphoreType.DMA,
            pltpu.SemaphoreType.DMA,
        ],
    )
    return scatter_k, combine_k


def kernel(x, Wr, W1, W2, W3, sW1, sW2, sW3):
    B, S, Dm = x.shape
    x_flat = x.reshape(T, D)

    aux, posn, pbb, tiles_i = pl.pallas_call(
        _plan_body,
        out_shape=(
            jax.ShapeDtypeStruct((8, 128), jnp.float32),
            jax.ShapeDtypeStruct((T, 8), jnp.int32),
            jax.ShapeDtypeStruct((T, 128), jnp.float32),
            jax.ShapeDtypeStruct((64, 8), jnp.int32),
        ),
    )(x_flat, Wr)

    scatter_k, combine_k = _sc_kernels()
    pos0 = posn[:, 0]
    pos1 = posn[:, 1]
    x_sorted = scatter_k(x_flat, pos0, pos1)

    buf = pl.pallas_call(
        _moe_body,
        grid_spec=pltpu.PrefetchScalarGridSpec(
            num_scalar_prefetch=1,
            grid=(NGRID,),
            in_specs=[
                pl.BlockSpec((TR, D), lambda i, tr: (tr[i, 2], 0)),
                pl.BlockSpec((TR, D), lambda i, tr: (jnp.maximum(i - NTS, 0), 0)),
                pl.BlockSpec((1, H, D), lambda i, tr: (jnp.minimum(tr[i, 0], E - 1), 0, 0)),
                pl.BlockSpec((1, H, D), lambda i, tr: (jnp.minimum(tr[i, 0], E - 1), 0, 0)),
                pl.BlockSpec((1, D, H), lambda i, tr: (jnp.minimum(tr[i, 0], E - 1), 0, 0)),
                pl.BlockSpec((1, H, D), lambda i, tr: (0, 0, 0)),
                pl.BlockSpec((1, H, D), lambda i, tr: (0, 0, 0)),
                pl.BlockSpec((1, D, H), lambda i, tr: (0, 0, 0)),
            ],
            out_specs=pl.BlockSpec((TR, D), lambda i, tr: (i, 0)),
        ),
        out_shape=jax.ShapeDtypeStruct((BUF_ROWS, D), jnp.float32),
    )(tiles_i, x_sorted, x_flat, W1, W3, W2, sW1, sW3, sW2)

    out = combine_k(buf, pos0, pos1, pbb)

    return out.reshape(B, S, Dm), aux[0, 0]


# speedup vs baseline: 1.1489x; 1.1489x over previous
import functools

import jax
import jax.numpy as jnp
from jax import lax
from jax.experimental import pallas as pl
from jax.experimental.pallas import tpu as pltpu
from jax.experimental.pallas import tpu_sc as plsc

E = 8
T = 2048
D = 768
H = 512
COEF = 0.01
NEG = -1e30
TR = 128
SORT_ROWS = 5120
NTS = SORT_ROWS // TR
NSH = T // TR
NGRID = NTS + NSH
BUF_ROWS = SORT_ROWS + T


def _cumsum_sub_excl(a):
    n = a.shape[0]
    incl = a
    s = 1
    while s < n:
        incl = incl + jnp.pad(incl, ((s, 0), (0, 0)))[:n]
        s *= 2
    return incl - a


def _cumsum_lane_excl(a):
    n = a.shape[1]
    a = jnp.pad(a, ((0, 0), (1, 0)))[:, :n]
    s = 1
    while s < n:
        a = a + jnp.pad(a, ((0, 0), (s, 0)))[:, :n]
        s *= 2
    return a


def _plan_body(x_ref, wr_ref, aux_ref, posn_ref, pbb_ref, tiles_ref):
    x = x_ref[...]
    wr = wr_ref[...]
    logits = lax.dot_general(x, wr, (((1,), (1,)), ((), ())),
                             preferred_element_type=jnp.float32)
    lane = lax.broadcasted_iota(jnp.int32, (T, E), 1)

    m = jnp.max(logits, axis=1, keepdims=True)
    ex = jnp.exp(logits - m)
    probs = ex / jnp.sum(ex, axis=1, keepdims=True)

    i1 = jnp.argmax(logits, axis=1).reshape(T, 1)
    oh1 = (lane == i1).astype(jnp.float32)
    p1 = jnp.max(probs, axis=1, keepdims=True)
    logits2 = jnp.where(lane == i1, NEG, logits)
    i2 = jnp.argmax(logits2, axis=1).reshape(T, 1)
    oh2 = (lane == i2).astype(jnp.float32)
    p2 = jnp.max(jnp.where(lane == i1, NEG, probs), axis=1, keepdims=True)

    density = jnp.mean(oh1, axis=0, keepdims=True)
    rpm = jnp.mean(probs, axis=0, keepdims=True)
    aux = COEF * jnp.sum(density * rpm) * E
    aux_ref[...] = jnp.full((8, 128), aux, dtype=jnp.float32)

    c0 = _cumsum_sub_excl(oh1)
    c1 = _cumsum_sub_excl(oh2)
    total0 = jnp.sum(oh1, axis=0, keepdims=True)
    total1 = jnp.sum(oh2, axis=0, keepdims=True)
    counts = total0 + total1
    cnt_al = jnp.floor((counts + (TR - 1)) / TR) * TR
    base = _cumsum_lane_excl(cnt_al)

    rank0 = jnp.sum(c0 * oh1, axis=1, keepdims=True)
    rank1 = jnp.sum((c1 + total0) * oh2, axis=1, keepdims=True)
    base0 = jnp.sum(base * oh1, axis=1, keepdims=True)
    base1 = jnp.sum(base * oh2, axis=1, keepdims=True)
    pos0f = base0 + rank0
    pos1f = base1 + rank1

    posn_ref[...] = (pos0f * (lane == 0) + pos1f * (lane == 1)).astype(jnp.int32)

    lane128 = lax.broadcasted_iota(jnp.int32, (T, 128), 1)
    pbb_ref[...] = jnp.where(lane128 < 16, p1,
                             jnp.where(lane128 < 32, p2, 0.0))

    s_i = lax.broadcasted_iota(jnp.int32, (64, E), 0).astype(jnp.float32) * TR
    hit = (s_i >= base).astype(jnp.float32)
    te_col = jnp.sum(hit, axis=1, keepdims=True) - 1.0
    total_al = jnp.sum(cnt_al, axis=1, keepdims=True)
    tv_col = (s_i[:, 0:1] < total_al).astype(jnp.float32)
    row_f = lax.broadcasted_iota(jnp.int32, (64, 1), 0).astype(jnp.float32)
    xi_col = jnp.where(tv_col > 0.0, row_f, float(NTS - 1))
    lane8b = lax.broadcasted_iota(jnp.int32, (64, 8), 1)
    tiles_ref[...] = (te_col * (lane8b == 0) + tv_col * (lane8b == 1)
                      + xi_col * (lane8b == 2)).astype(jnp.int32)


def _mlp(x, w1, w3, w2):
    h1 = lax.dot_general(x, w1, (((1,), (1,)), ((), ())),
                         preferred_element_type=jnp.float32)
    h3 = lax.dot_general(x, w3, (((1,), (1,)), ((), ())),
                         preferred_element_type=jnp.float32)
    g = h1 * jax.nn.sigmoid(h1) * h3
    return lax.dot_general(g, w2, (((1,), (1,)), ((), ())),
                           preferred_element_type=jnp.float32)


def _shared_body(x_ref, w1_ref, w3_ref, w2_ref, out_ref):
    out_ref[...] = _mlp(x_ref[...], w1_ref[0], w3_ref[0], w2_ref[0])


def _routed_body(tr_ref, xs_ref, w1_ref, w3_ref, w2_ref, out_ref):
    i = pl.program_id(0)

    @pl.when(tr_ref[i, 1] == 1)
    def _():
        out_ref[...] = _mlp(xs_ref[...], w1_ref[0], w3_ref[0], w2_ref[0])


def _scatter_body(x_hbm, pos0_hbm, pos1_hbm, xs_hbm, idx0_v, idx1_v,
                  rows_v, sem):
    wid = lax.axis_index("s") * 2 + lax.axis_index("c")
    tok0 = wid * 64
    pltpu.sync_copy(pos0_hbm.at[pl.ds(tok0, 64)], idx0_v)
    pltpu.sync_copy(pos1_hbm.at[pl.ds(tok0, 64)], idx1_v)
    pltpu.sync_copy(x_hbm.at[pl.ds(tok0, 64)], rows_v)
    cp0 = pltpu.async_copy(rows_v, xs_hbm.at[idx0_v], sem)
    cp1 = pltpu.async_copy(rows_v, xs_hbm.at[idx1_v], sem)
    cp0.wait()
    cp1.wait()


def _combine_body(buf_hbm, bufsh_hbm, pos0_hbm, pos1_hbm, pbb_hbm, out_hbm,
                  idx0a_v, idx1a_v, idx0b_v, idx1b_v,
                  r0a_v, r1a_v, r0b_v, r1b_v, acc_v, pb_v, sem_a, sem_b):
    wid = lax.axis_index("s") * 2 + lax.axis_index("c")
    tok0 = wid * 64
    pltpu.sync_copy(pos0_hbm.at[pl.ds(tok0, 32)], idx0a_v)
    pltpu.sync_copy(pos1_hbm.at[pl.ds(tok0, 32)], idx1a_v)
    pltpu.sync_copy(pos0_hbm.at[pl.ds(tok0 + 32, 32)], idx0b_v)
    pltpu.sync_copy(pos1_hbm.at[pl.ds(tok0 + 32, 32)], idx1b_v)
    cps = [pltpu.async_copy(buf_hbm.at[idx0a_v], r0a_v, sem_a),
           pltpu.async_copy(buf_hbm.at[idx1a_v], r1a_v, sem_a),
           pltpu.async_copy(buf_hbm.at[idx0b_v], r0b_v, sem_b),
           pltpu.async_copy(buf_hbm.at[idx1b_v], r1b_v, sem_b)]

    for b, r0_v, r1_v in ((0, r0a_v, r1a_v), (1, r0b_v, r1b_v)):
        t0 = tok0 + b * 32
        pltpu.sync_copy(bufsh_hbm.at[pl.ds(t0, 32)], acc_v)
        pltpu.sync_copy(pbb_hbm.at[pl.ds(t0, 32)], pb_v)
        cps[2 * b].wait()
        cps[2 * b + 1].wait()

        def body(it, carry):
            for dt in range(4):
                t = it * 4 + dt
                a1 = pb_v[t, pl.ds(0, 16)]
                a2 = pb_v[t, pl.ds(16, 16)]
                for ch in range(48):
                    sl = pl.ds(ch * 16, 16)
                    acc_v[t, sl] = (acc_v[t, sl] + a1 * r0_v[t, sl]
                                    + a2 * r1_v[t, sl])
            return carry

        lax.fori_loop(0, 8, body, 0)
        pltpu.sync_copy(acc_v, out_hbm.at[pl.ds(t0, 32)])


@functools.lru_cache(maxsize=1)
def _sc_kernels():
    mesh = plsc.VectorSubcoreMesh(core_axis_name="c", subcore_axis_name="s")
    scatter_k = pl.kernel(
        _scatter_body,
        out_type=jax.ShapeDtypeStruct((SORT_ROWS, D), jnp.float32),
        mesh=mesh,
        scratch_types=[
            pltpu.VMEM((64,), jnp.int32),
            pltpu.VMEM((64,), jnp.int32),
            pltpu.VMEM((64, D), jnp.float32),
            pltpu.SemaphoreType.DMA,
        ],
    )
    combine_k = pl.kernel(
        _combine_body,
        out_type=jax.ShapeDtypeStruct((T, D), jnp.float32),
        mesh=mesh,
        scratch_types=[
            pltpu.VMEM((32,), jnp.int32),
            pltpu.VMEM((32,), jnp.int32),
            pltpu.VMEM((32,), jnp.int32),
            pltpu.VMEM((32,), jnp.int32),
            pltpu.VMEM((32, D), jnp.float32),
            pltpu.VMEM((32, D), jnp.float32),
            pltpu.VMEM((32, D), jnp.float32),
            pltpu.VMEM((32, D), jnp.float32),
            pltpu.VMEM((32, D), jnp.float32),
            pltpu.VMEM((32, 128), jnp.float32),
            pltpu.SemaphoreType.DMA,
            pltpu.SemaphoreType.DMA,
        ],
    )
    return scatter_k, combine_k


def kernel(x, Wr, W1, W2, W3, sW1, sW2, sW3):
    B, S, Dm = x.shape
    x_flat = x.reshape(T, D)

    aux, posn, pbb, tiles_i = pl.pallas_call(
        _plan_body,
        out_shape=(
            jax.ShapeDtypeStruct((8, 128), jnp.float32),
            jax.ShapeDtypeStruct((T, 8), jnp.int32),
            jax.ShapeDtypeStruct((T, 128), jnp.float32),
            jax.ShapeDtypeStruct((64, 8), jnp.int32),
        ),
    )(x_flat, Wr)

    scatter_k, combine_k = _sc_kernels()
    pos0 = posn[:, 0]
    pos1 = posn[:, 1]
    x_sorted = scatter_k(x_flat, pos0, pos1)

    bufsh = pl.pallas_call(
        _shared_body,
        in_specs=[
            pl.BlockSpec((T, D), lambda: (0, 0)),
            pl.BlockSpec((1, H, D), lambda: (0, 0, 0)),
            pl.BlockSpec((1, H, D), lambda: (0, 0, 0)),
            pl.BlockSpec((1, D, H), lambda: (0, 0, 0)),
        ],
        out_specs=pl.BlockSpec((T, D), lambda: (0, 0)),
        out_shape=jax.ShapeDtypeStruct((T, D), jnp.float32),
    )(x_flat, sW1, sW3, sW2)

    buf = pl.pallas_call(
        _routed_body,
        grid_spec=pltpu.PrefetchScalarGridSpec(
            num_scalar_prefetch=1,
            grid=(NTS,),
            in_specs=[
                pl.BlockSpec((TR, D), lambda i, tr: (tr[i, 2], 0)),
                pl.BlockSpec((1, H, D), lambda i, tr: (jnp.minimum(tr[i, 0], E - 1), 0, 0)),
                pl.BlockSpec((1, H, D), lambda i, tr: (jnp.minimum(tr[i, 0], E - 1), 0, 0)),
                pl.BlockSpec((1, D, H), lambda i, tr: (jnp.minimum(tr[i, 0], E - 1), 0, 0)),
            ],
            out_specs=pl.BlockSpec((TR, D), lambda i, tr: (i, 0)),
        ),
        out_shape=jax.ShapeDtypeStruct((SORT_ROWS, D), jnp.float32),
    )(tiles_i, x_sorted, W1, W3, W2)

    out = combine_k(buf, bufsh, pos0, pos1, pbb)

    return out.reshape(B, S, Dm), aux[0, 0]
